# Initial kernel scaffold; baseline (speedup 1.0000x reference)
import jax
import jax.numpy as jnp
from jax.experimental import pallas as pl


def kernel(inputs_real, inputs_imag, sc_ind, c_diag_real, c_diag_imag):
    raise NotImplementedError("write your pallas kernel here")


# speedup vs baseline: 32.6522x; 32.6522x over previous
import jax
import jax.numpy as jnp
from jax.experimental import pallas as pl

_B, _R, _A, _S, _F = 16, 2, 4, 14, 4096
_N = _B * _R * _A
_BK = 8


def _demash_body(ir_ref, ii_ref, wr_ref, wi_ref, out_ref):
    ir = ir_ref[...]
    ii = ii_ref[...]
    wr = wr_ref[...][None]
    wi = wi_ref[...][None]
    out_ref[0] = ir * wr + ii * wi
    out_ref[1] = ii * wr - ir * wi


def kernel(inputs_real, inputs_imag, sc_ind, c_diag_real, c_diag_imag):
    n_sc = sc_ind.shape[0]
    ir = inputs_real.reshape(_N, _S, _F)
    ii = inputs_imag.reshape(_N, _S, _F)
    cr = c_diag_real.reshape(_S, n_sc)
    ci = c_diag_imag.reshape(_S, n_sc)
    wr = jnp.zeros((_S, _F), jnp.float32).at[:, sc_ind].set(cr)
    wi = jnp.zeros((_S, _F), jnp.float32).at[:, sc_ind].set(ci)

    out = pl.pallas_call(
        _demash_body,
        grid=(_N // _BK,),
        in_specs=[
            pl.BlockSpec((_BK, _S, _F), lambda i: (i, 0, 0)),
            pl.BlockSpec((_BK, _S, _F), lambda i: (i, 0, 0)),
            pl.BlockSpec((_S, _F), lambda i: (0, 0)),
            pl.BlockSpec((_S, _F), lambda i: (0, 0)),
        ],
        out_specs=pl.BlockSpec((2, _BK, _S, _F), lambda i: (0, i, 0, 0)),
        out_shape=jax.ShapeDtypeStruct((2, _N, _S, _F), jnp.float32),
    )(ir, ii, wr, wi)
    return out.reshape(2, _B, _R, _A, _S, _F)
